# baseline (speedup 1.0000x reference)
import functools

import jax
import jax.numpy as jnp
from jax import lax
from jax.experimental import pallas as pl
from jax.experimental.pallas import tpu as pltpu
from jax.experimental.pallas import tpu_sc as plsc

BATCH = 16384
DIM = 64
NW = 32
BPW = BATCH // NW
CHUNK = 128
NCH = BPW // CHUNK
G = 16
NG = CHUNK // G


def _nrsqrt(x):
    i = plsc.bitcast(x, jnp.int32)
    i = jnp.int32(0x5F3759DF) - lax.shift_right_arithmetic(i, jnp.int32(1))
    y = plsc.bitcast(i, jnp.float32)
    for _ in range(3):
        y = y * (1.5 - 0.5 * x * y * y)
    return y


def _body(hrow_r, hpar_r, trow_r, tpar_r, rrow_r, rpar_r,
          entity_hbm, relation_hbm, out_hbm,
          idx_h, par_h, idx_t, par_t, idx_r, par_r,
          h_bufs, t_bufs, r_bufs, out_v, sems):
    wid = lax.axis_index("s") * 2 + lax.axis_index("c")
    base = pl.multiple_of(wid * BPW, BPW)
    irow = pl.multiple_of(wid * NCH, NCH)

    pltpu.sync_copy(hrow_r.at[pl.ds(irow, NCH)], idx_h)
    pltpu.sync_copy(hpar_r.at[pl.ds(irow, NCH)], par_h)
    pltpu.sync_copy(trow_r.at[pl.ds(irow, NCH)], idx_t)
    pltpu.sync_copy(tpar_r.at[pl.ds(irow, NCH)], par_t)
    pltpu.sync_copy(rrow_r.at[pl.ds(irow, NCH)], idx_r)
    pltpu.sync_copy(rpar_r.at[pl.ds(irow, NCH)], par_r)

    def fire(c):
        buf = c % 2
        return [
            pltpu.async_copy(entity_hbm.at[idx_h.at[c]], h_bufs[buf], sems[buf]),
            pltpu.async_copy(entity_hbm.at[idx_t.at[c]], t_bufs[buf], sems[buf]),
            pltpu.async_copy(relation_hbm.at[idx_r.at[c]], r_bufs[buf], sems[buf]),
        ]

    lane = lax.iota(jnp.int32, 16)
    zero = jnp.zeros((16,), jnp.float32)
    sixty_four = jnp.full((16,), DIM, jnp.int32)

    inflight = fire(0)
    for c in range(NCH):
        if c + 1 < NCH:
            next_copies = fire(c + 1)
        else:
            next_copies = []
        for cp in inflight:
            cp.wait()
        inflight = next_copies

        buf = c % 2
        h_buf, t_buf, r_buf = h_bufs[buf], t_bufs[buf], r_bufs[buf]

        def group(g, carry, c=c, h_buf=h_buf, t_buf=t_buf, r_buf=r_buf):
            gbase = pl.multiple_of(g * G, G)
            rows = gbase + lane
            colh0 = par_h[c, pl.ds(gbase, G)] * sixty_four
            colt0 = par_t[c, pl.ds(gbase, G)] * sixty_four
            colr0 = par_r[c, pl.ds(gbase, G)] * sixty_four
            hh = tt = rr = hr = ht = rt = zero
            for j in range(DIM):
                h = plsc.load_gather(h_buf, [rows, colh0 + j])
                t = plsc.load_gather(t_buf, [rows, colt0 + j])
                r = plsc.load_gather(r_buf, [rows, colr0 + j])
                hh = hh + h * h
                tt = tt + t * t
                rr = rr + r * r
                hr = hr + h * r
                ht = ht + h * t
                rt = rt + r * t
            a = _nrsqrt(jnp.maximum(hh, 1e-24))
            b = _nrsqrt(jnp.maximum(tt, 1e-24))
            s2 = (hh * a * a + rr + tt * b * b
                  + 2.0 * (hr * a - ht * (a * b) - rt * b))
            s2 = jnp.maximum(s2, 0.0)
            score = s2 * _nrsqrt(jnp.maximum(s2, 1e-30))
            out_v[pl.ds(c * CHUNK + gbase, G)] = score
            return carry

        lax.fori_loop(0, NG, group, 0)

    pltpu.sync_copy(out_v, out_hbm.at[pl.ds(base, BPW)])


_sc_kernel = functools.partial(
    pl.kernel,
    mesh=plsc.VectorSubcoreMesh(core_axis_name="c", subcore_axis_name="s"),
    compiler_params=pltpu.CompilerParams(
        needs_layout_passes=False, use_tc_tiling_on_sc=True),
    out_type=jax.ShapeDtypeStruct((BATCH,), jnp.float32),
    scratch_types=[
        pltpu.VMEM((NCH, CHUNK), jnp.int32),
        pltpu.VMEM((NCH, CHUNK), jnp.int32),
        pltpu.VMEM((NCH, CHUNK), jnp.int32),
        pltpu.VMEM((NCH, CHUNK), jnp.int32),
        pltpu.VMEM((NCH, CHUNK), jnp.int32),
        pltpu.VMEM((NCH, CHUNK), jnp.int32),
        [pltpu.VMEM((CHUNK, 2 * DIM), jnp.float32) for _ in range(2)],
        [pltpu.VMEM((CHUNK, 2 * DIM), jnp.float32) for _ in range(2)],
        [pltpu.VMEM((CHUNK, 2 * DIM), jnp.float32) for _ in range(2)],
        pltpu.VMEM((BPW,), jnp.float32),
        [pltpu.SemaphoreType.DMA for _ in range(2)],
    ],
)(_body)


def kernel(heads, relations, tails, entity_table, relation_table):
    heads = heads.astype(jnp.int32)
    relations = relations.astype(jnp.int32)
    tails = tails.astype(jnp.int32)
    shape2 = (BATCH // CHUNK, CHUNK)
    return _sc_kernel(
        (heads >> 1).reshape(shape2), (heads & 1).reshape(shape2),
        (tails >> 1).reshape(shape2), (tails & 1).reshape(shape2),
        (relations >> 1).reshape(shape2), (relations & 1).reshape(shape2),
        entity_table.reshape(-1, 2 * DIM),
        relation_table.reshape(-1, 2 * DIM),
    )


# speedup vs baseline: 1.6240x; 1.6240x over previous
import functools

import jax
import jax.numpy as jnp
from jax import lax
from jax.experimental import pallas as pl
from jax.experimental.pallas import tpu as pltpu
from jax.experimental.pallas import tpu_sc as plsc

BATCH = 16384
DIM = 64
NW = 32
BPW = BATCH // NW
CHUNK = 128
NCH = BPW // CHUNK
G = 16
NG = CHUNK // G


def _nrsqrt(x):
    i = plsc.bitcast(x, jnp.int32)
    i = jnp.int32(0x5F3759DF) - lax.shift_right_arithmetic(i, jnp.int32(1))
    y = plsc.bitcast(i, jnp.float32)
    for _ in range(3):
        y = y * (1.5 - 0.5 * x * y * y)
    return y


def _body(heads_r, rels_r, tails_r, entity_hbm, relation_hbm, out_hbm,
          idx_h, idx_r, idx_t, h_bufs, t_bufs, r_bufs, out_v, sems):
    wid = lax.axis_index("s") * 2 + lax.axis_index("c")
    base = pl.multiple_of(wid * BPW, BPW)
    irow = pl.multiple_of(wid * NCH, NCH)

    pltpu.sync_copy(heads_r.at[pl.ds(irow, NCH)], idx_h)
    pltpu.sync_copy(rels_r.at[pl.ds(irow, NCH)], idx_r)
    pltpu.sync_copy(tails_r.at[pl.ds(irow, NCH)], idx_t)

    def fire(c):
        buf = c % 2

        def fg(g, carry):
            gbase = pl.multiple_of(g * G, G)
            vh = idx_h[c, pl.ds(gbase, G)]
            vt = idx_t[c, pl.ds(gbase, G)]
            vr = idx_r[c, pl.ds(gbase, G)]
            for l in range(G):
                pltpu.async_copy(
                    entity_hbm.at[vh[l]], h_bufs[buf].at[gbase + l], sems[buf])
                pltpu.async_copy(
                    entity_hbm.at[vt[l]], t_bufs[buf].at[gbase + l], sems[buf])
                pltpu.async_copy(
                    relation_hbm.at[vr[l]], r_bufs[buf].at[gbase + l], sems[buf])
            return carry

        lax.fori_loop(0, NG, fg, 0)

    def drain(c):
        buf = c % 2
        pltpu.make_async_copy(
            entity_hbm.at[pl.ds(0, CHUNK)], h_bufs[buf], sems[buf]).wait()
        pltpu.make_async_copy(
            entity_hbm.at[pl.ds(0, CHUNK)], t_bufs[buf], sems[buf]).wait()
        pltpu.make_async_copy(
            relation_hbm.at[pl.ds(0, CHUNK)], r_bufs[buf], sems[buf]).wait()

    lane = lax.iota(jnp.int32, 16)
    zero = jnp.zeros((16,), jnp.float32)

    fire(0)
    for c in range(NCH):
        if c + 1 < NCH:
            fire(c + 1)
        drain(c)

        buf = c % 2
        h_buf, t_buf, r_buf = h_bufs[buf], t_bufs[buf], r_bufs[buf]

        def group(g, carry, c=c, h_buf=h_buf, t_buf=t_buf, r_buf=r_buf):
            gbase = pl.multiple_of(g * G, G)
            rows = gbase + lane
            hh = tt = rr = hr = ht = rt = zero
            for j in range(DIM):
                col = jnp.full((16,), j, jnp.int32)
                h = plsc.load_gather(h_buf, [rows, col])
                t = plsc.load_gather(t_buf, [rows, col])
                r = plsc.load_gather(r_buf, [rows, col])
                hh = hh + h * h
                tt = tt + t * t
                rr = rr + r * r
                hr = hr + h * r
                ht = ht + h * t
                rt = rt + r * t
            a = _nrsqrt(jnp.maximum(hh, 1e-24))
            b = _nrsqrt(jnp.maximum(tt, 1e-24))
            s2 = (hh * a * a + rr + tt * b * b
                  + 2.0 * (hr * a - ht * (a * b) - rt * b))
            s2 = jnp.maximum(s2, 0.0)
            score = s2 * _nrsqrt(jnp.maximum(s2, 1e-30))
            out_v[pl.ds(c * CHUNK + gbase, G)] = score
            return carry

        lax.fori_loop(0, NG, group, 0)

    pltpu.sync_copy(out_v, out_hbm.at[pl.ds(base, BPW)])


_sc_kernel = functools.partial(
    pl.kernel,
    mesh=plsc.VectorSubcoreMesh(core_axis_name="c", subcore_axis_name="s"),
    compiler_params=pltpu.CompilerParams(
        needs_layout_passes=False, use_tc_tiling_on_sc=True),
    out_type=jax.ShapeDtypeStruct((BATCH,), jnp.float32),
    scratch_types=[
        pltpu.VMEM((NCH, CHUNK), jnp.int32),
        pltpu.VMEM((NCH, CHUNK), jnp.int32),
        pltpu.VMEM((NCH, CHUNK), jnp.int32),
        [pltpu.VMEM((CHUNK, DIM), jnp.float32) for _ in range(2)],
        [pltpu.VMEM((CHUNK, DIM), jnp.float32) for _ in range(2)],
        [pltpu.VMEM((CHUNK, DIM), jnp.float32) for _ in range(2)],
        pltpu.VMEM((BPW,), jnp.float32),
        [pltpu.SemaphoreType.DMA for _ in range(2)],
    ],
)(_body)


def kernel(heads, relations, tails, entity_table, relation_table):
    shape2 = (BATCH // CHUNK, CHUNK)
    return _sc_kernel(
        heads.astype(jnp.int32).reshape(shape2),
        relations.astype(jnp.int32).reshape(shape2),
        tails.astype(jnp.int32).reshape(shape2),
        entity_table,
        relation_table,
    )


# speedup vs baseline: 2.2748x; 1.4008x over previous
import functools

import jax
import jax.numpy as jnp
from jax import lax
from jax.experimental import pallas as pl
from jax.experimental.pallas import tpu as pltpu
from jax.experimental.pallas import tpu_sc as plsc

BATCH = 16384
DIM = 64
NW = 32
BPW = BATCH // NW
CHUNK = 128
NCH = BPW // CHUNK
G = 16
NG = CHUNK // G


def _nrsqrt(x):
    i = plsc.bitcast(x, jnp.int32)
    i = jnp.int32(0x5F3759DF) - lax.shift_right_arithmetic(i, jnp.int32(1))
    y = plsc.bitcast(i, jnp.float32)
    for _ in range(3):
        y = y * (1.5 - 0.5 * x * y * y)
    return y


def _body(heads_r, rels_r, tails_r, entity_hbm, relation_hbm, out_hbm,
          idx_h, idx_r, idx_t, h_bufs, t_bufs, r_bufs, out_v, sems):
    wid = lax.axis_index("s") * 2 + lax.axis_index("c")
    base = pl.multiple_of(wid * BPW, BPW)
    irow = pl.multiple_of(wid * NCH, NCH)

    pltpu.sync_copy(heads_r.at[pl.ds(irow, NCH)], idx_h)
    pltpu.sync_copy(rels_r.at[pl.ds(irow, NCH)], idx_r)
    pltpu.sync_copy(tails_r.at[pl.ds(irow, NCH)], idx_t)

    def fire(c):
        buf = c % 2

        def fg(g, carry):
            gbase = pl.multiple_of(g * G, G)
            vh = idx_h[c, pl.ds(gbase, G)]
            vt = idx_t[c, pl.ds(gbase, G)]
            vr = idx_r[c, pl.ds(gbase, G)]
            for l in range(G):
                dq, ds_ = g * 2 + l // 8, l % 8
                pltpu.async_copy(
                    entity_hbm.at[vh[l] >> 3, vh[l] & 7],
                    h_bufs[buf].at[dq, ds_], sems[buf])
                pltpu.async_copy(
                    entity_hbm.at[vt[l] >> 3, vt[l] & 7],
                    t_bufs[buf].at[dq, ds_], sems[buf])
                pltpu.async_copy(
                    relation_hbm.at[vr[l] >> 3, vr[l] & 7],
                    r_bufs[buf].at[dq, ds_], sems[buf])
            return carry

        lax.fori_loop(0, NG, fg, 0)

    def drain(c):
        buf = c % 2
        pltpu.make_async_copy(
            entity_hbm.at[pl.ds(0, CHUNK // 8)], h_bufs[buf], sems[buf]).wait()
        pltpu.make_async_copy(
            entity_hbm.at[pl.ds(0, CHUNK // 8)], t_bufs[buf], sems[buf]).wait()
        pltpu.make_async_copy(
            relation_hbm.at[pl.ds(0, CHUNK // 8)], r_bufs[buf], sems[buf]).wait()

    lane = lax.iota(jnp.int32, 16)
    zero = jnp.zeros((16,), jnp.float32)

    fire(0)
    for c in range(NCH):
        if c + 1 < NCH:
            fire(c + 1)
        drain(c)

        buf = c % 2
        h_buf, t_buf, r_buf = h_bufs[buf], t_bufs[buf], r_bufs[buf]

        def group(g, carry, c=c, h_buf=h_buf, t_buf=t_buf, r_buf=r_buf):
            gbase = pl.multiple_of(g * G, G)
            rows = gbase + lane
            rq = lax.shift_right_logical(rows, 3)
            rs = lax.bitwise_and(rows, 7)
            hh = tt = rr = hr = ht = rt = zero
            for j in range(DIM):
                col = jnp.full((16,), j, jnp.int32)
                h = plsc.load_gather(h_buf, [rq, rs, col])
                t = plsc.load_gather(t_buf, [rq, rs, col])
                r = plsc.load_gather(r_buf, [rq, rs, col])
                hh = hh + h * h
                tt = tt + t * t
                rr = rr + r * r
                hr = hr + h * r
                ht = ht + h * t
                rt = rt + r * t
            a = _nrsqrt(jnp.maximum(hh, 1e-24))
            b = _nrsqrt(jnp.maximum(tt, 1e-24))
            s2 = (hh * a * a + rr + tt * b * b
                  + 2.0 * (hr * a - ht * (a * b) - rt * b))
            s2 = jnp.maximum(s2, 0.0)
            score = s2 * _nrsqrt(jnp.maximum(s2, 1e-30))
            out_v[pl.ds(c * CHUNK + gbase, G)] = score
            return carry

        lax.fori_loop(0, NG, group, 0)

    pltpu.sync_copy(out_v, out_hbm.at[pl.ds(base, BPW)])


_sc_kernel = functools.partial(
    pl.kernel,
    mesh=plsc.VectorSubcoreMesh(core_axis_name="c", subcore_axis_name="s"),
    compiler_params=pltpu.CompilerParams(
        needs_layout_passes=False, use_tc_tiling_on_sc=True),
    out_type=jax.ShapeDtypeStruct((BATCH,), jnp.float32),
    scratch_types=[
        pltpu.VMEM((NCH, CHUNK), jnp.int32),
        pltpu.VMEM((NCH, CHUNK), jnp.int32),
        pltpu.VMEM((NCH, CHUNK), jnp.int32),
        [pltpu.VMEM((CHUNK // 8, 8, DIM), jnp.float32) for _ in range(2)],
        [pltpu.VMEM((CHUNK // 8, 8, DIM), jnp.float32) for _ in range(2)],
        [pltpu.VMEM((CHUNK // 8, 8, DIM), jnp.float32) for _ in range(2)],
        pltpu.VMEM((BPW,), jnp.float32),
        [pltpu.SemaphoreType.DMA for _ in range(2)],
    ],
)(_body)


def kernel(heads, relations, tails, entity_table, relation_table):
    shape2 = (BATCH // CHUNK, CHUNK)
    return _sc_kernel(
        heads.astype(jnp.int32).reshape(shape2),
        relations.astype(jnp.int32).reshape(shape2),
        tails.astype(jnp.int32).reshape(shape2),
        entity_table.reshape(-1, 8, DIM),
        relation_table.reshape(-1, 8, DIM),
    )


# speedup vs baseline: 2.2786x; 1.0017x over previous
import functools

import jax
import jax.numpy as jnp
from jax import lax
from jax.experimental import pallas as pl
from jax.experimental.pallas import tpu as pltpu
from jax.experimental.pallas import tpu_sc as plsc

BATCH = 16384
DIM = 64
NW = 32
BPW = BATCH // NW
CHUNK = 128
NCH = BPW // CHUNK
G = 16
NG = CHUNK // G


def _nrsqrt(x):
    i = plsc.bitcast(x, jnp.int32)
    i = jnp.int32(0x5F3759DF) - lax.shift_right_arithmetic(i, jnp.int32(1))
    y = plsc.bitcast(i, jnp.float32)
    for _ in range(3):
        y = y * (1.5 - 0.5 * x * y * y)
    return y


def _body(heads_r, rels_r, tails_r, entity_hbm, relation_hbm, out_hbm,
          idx_h, idx_r, idx_t, h_bufs, t_bufs, r_bufs, out_v,
          sems_h, sems_t, sems_r):
    wid = lax.axis_index("s") * 2 + lax.axis_index("c")
    base = pl.multiple_of(wid * BPW, BPW)
    irow = pl.multiple_of(wid * NCH, NCH)

    pltpu.sync_copy(heads_r.at[pl.ds(irow, NCH)], idx_h)
    pltpu.sync_copy(rels_r.at[pl.ds(irow, NCH)], idx_r)
    pltpu.sync_copy(tails_r.at[pl.ds(irow, NCH)], idx_t)

    def fire(c):
        buf = c % 2

        def fg(g, carry):
            gbase = pl.multiple_of(g * G, G)
            vh = idx_h[c, pl.ds(gbase, G)]
            vt = idx_t[c, pl.ds(gbase, G)]
            vr = idx_r[c, pl.ds(gbase, G)]
            for l in range(G):
                dq, ds_ = g * 2 + l // 8, l % 8
                pltpu.async_copy(
                    entity_hbm.at[vh[l] >> 3, vh[l] & 7],
                    h_bufs[buf].at[dq, ds_], sems_h[buf])
                pltpu.async_copy(
                    entity_hbm.at[vt[l] >> 3, vt[l] & 7],
                    t_bufs[buf].at[dq, ds_], sems_t[buf])
                pltpu.async_copy(
                    relation_hbm.at[vr[l] >> 3, vr[l] & 7],
                    r_bufs[buf].at[dq, ds_], sems_r[buf])
            return carry

        lax.fori_loop(0, NG, fg, 0)

    def drain(c):
        buf = c % 2
        pltpu.make_async_copy(
            entity_hbm.at[pl.ds(0, CHUNK // 8)], h_bufs[buf], sems_h[buf]).wait()
        pltpu.make_async_copy(
            entity_hbm.at[pl.ds(0, CHUNK // 8)], t_bufs[buf], sems_t[buf]).wait()
        pltpu.make_async_copy(
            relation_hbm.at[pl.ds(0, CHUNK // 8)], r_bufs[buf], sems_r[buf]).wait()

    lane = lax.iota(jnp.int32, 16)
    zero = jnp.zeros((16,), jnp.float32)

    fire(0)
    for c in range(NCH):
        if c + 1 < NCH:
            fire(c + 1)
        drain(c)

        buf = c % 2
        h_buf, t_buf, r_buf = h_bufs[buf], t_bufs[buf], r_bufs[buf]

        def group(g, carry, c=c, h_buf=h_buf, t_buf=t_buf, r_buf=r_buf):
            gbase = pl.multiple_of(g * G, G)
            rows = gbase + lane
            rq = lax.shift_right_logical(rows, 3)
            rs = lax.bitwise_and(rows, 7)
            hh = tt = rr = hr = ht = rt = zero
            for j in range(DIM):
                col = jnp.full((16,), j, jnp.int32)
                h = plsc.load_gather(h_buf, [rq, rs, col])
                t = plsc.load_gather(t_buf, [rq, rs, col])
                r = plsc.load_gather(r_buf, [rq, rs, col])
                hh = hh + h * h
                tt = tt + t * t
                rr = rr + r * r
                hr = hr + h * r
                ht = ht + h * t
                rt = rt + r * t
            a = _nrsqrt(jnp.maximum(hh, 1e-24))
            b = _nrsqrt(jnp.maximum(tt, 1e-24))
            s2 = (hh * a * a + rr + tt * b * b
                  + 2.0 * (hr * a - ht * (a * b) - rt * b))
            s2 = jnp.maximum(s2, 0.0)
            score = s2 * _nrsqrt(jnp.maximum(s2, 1e-30))
            out_v[pl.ds(c * CHUNK + gbase, G)] = score
            return carry

        lax.fori_loop(0, NG, group, 0)

    pltpu.sync_copy(out_v, out_hbm.at[pl.ds(base, BPW)])


_sc_kernel = functools.partial(
    pl.kernel,
    mesh=plsc.VectorSubcoreMesh(core_axis_name="c", subcore_axis_name="s"),
    compiler_params=pltpu.CompilerParams(
        needs_layout_passes=False, use_tc_tiling_on_sc=True),
    out_type=jax.ShapeDtypeStruct((BATCH,), jnp.float32),
    scratch_types=[
        pltpu.VMEM((NCH, CHUNK), jnp.int32),
        pltpu.VMEM((NCH, CHUNK), jnp.int32),
        pltpu.VMEM((NCH, CHUNK), jnp.int32),
        [pltpu.VMEM((CHUNK // 8, 8, DIM), jnp.float32) for _ in range(2)],
        [pltpu.VMEM((CHUNK // 8, 8, DIM), jnp.float32) for _ in range(2)],
        [pltpu.VMEM((CHUNK // 8, 8, DIM), jnp.float32) for _ in range(2)],
        pltpu.VMEM((BPW,), jnp.float32),
        [pltpu.SemaphoreType.DMA for _ in range(2)],
        [pltpu.SemaphoreType.DMA for _ in range(2)],
        [pltpu.SemaphoreType.DMA for _ in range(2)],
    ],
)(_body)


def kernel(heads, relations, tails, entity_table, relation_table):
    shape2 = (BATCH // CHUNK, CHUNK)
    return _sc_kernel(
        heads.astype(jnp.int32).reshape(shape2),
        relations.astype(jnp.int32).reshape(shape2),
        tails.astype(jnp.int32).reshape(shape2),
        entity_table.reshape(-1, 8, DIM),
        relation_table.reshape(-1, 8, DIM),
    )
